# baseline (speedup 1.0000x reference)
import functools

import jax
import jax.numpy as jnp
import numpy as np
from jax import lax
from jax.experimental import pallas as pl
from jax.experimental.pallas import tpu as pltpu
from jax.experimental.pallas import tpu_sc as plsc

R = 128
N = 100000
NC = 2
NS = 16
NW = NC * NS
ROWS_PER_W = R // NW
CHUNK = 10000
NCHUNK = N // CHUNK
LANES = 16

def _perm_np():
    try:
        with jax.default_device(jax.local_devices(backend="cpu")[0]):
            p = jax.random.permutation(jax.random.key(42), N)
            return np.asarray(p, dtype=np.int32)
    except Exception:
        return None


_PERM = _perm_np()


def _perm_idx():
    if _PERM is not None:
        return jnp.asarray(_PERM)
    return jax.random.permutation(jax.random.key(42), N).astype(jnp.int32)


def _sc_body(in_hbm, idx_hbm, out_hbm, row_v, idx_v, out_v):
    wid = lax.axis_index("s") * NC + lax.axis_index("c")

    def row_body(k, carry):
        r = wid * ROWS_PER_W + k
        pltpu.sync_copy(in_hbm.at[pl.ds(r * N, N)], row_v)

        def chunk_body(j, carry):
            pltpu.sync_copy(idx_hbm.at[pl.ds(j * CHUNK, CHUNK)], idx_v)

            def vec_body(v, carry):
                iv = idx_v[pl.ds(v * LANES, LANES)]
                out_v[pl.ds(v * LANES, LANES)] = plsc.load_gather(row_v, [iv])
                return carry

            lax.fori_loop(0, CHUNK // LANES, vec_body, carry)
            pltpu.sync_copy(out_v, out_hbm.at[pl.ds(r * N + j * CHUNK, CHUNK)])
            return carry

        return lax.fori_loop(0, NCHUNK, chunk_body, carry)

    lax.fori_loop(0, ROWS_PER_W, row_body, 0)


_shuffle_sc = pl.kernel(
    _sc_body,
    out_type=jax.ShapeDtypeStruct((R * N,), jnp.float32),
    mesh=plsc.VectorSubcoreMesh(
        core_axis_name="c", subcore_axis_name="s", num_cores=NC, num_subcores=NS
    ),
    scratch_types=[
        pltpu.VMEM((N,), jnp.float32),
        pltpu.VMEM((CHUNK,), jnp.int32),
        pltpu.VMEM((CHUNK,), jnp.float32),
    ],
    compiler_params=pltpu.CompilerParams(needs_layout_passes=False),
)


@jax.jit
def _run(inputs):
    idx = _perm_idx()
    flat = _shuffle_sc(inputs.reshape(R * N), idx)
    return flat.reshape(R, N)


def kernel(inputs):
    return _run(inputs)


# speedup vs baseline: 1.1956x; 1.1956x over previous
import jax
import jax.numpy as jnp
import numpy as np
from jax import lax
from jax.experimental import pallas as pl
from jax.experimental.pallas import tpu as pltpu
from jax.experimental.pallas import tpu_sc as plsc

R = 128
N = 100000
NC = 2
NS = 16
NW = NC * NS
ROWS_PER_W = R // NW
CHUNK = 2000
NCHUNK = N // CHUNK
LANES = 16
STEPS = CHUNK // LANES


def _perm_np():
    try:
        with jax.default_device(jax.local_devices(backend="cpu")[0]):
            p = jax.random.permutation(jax.random.key(42), N)
            return np.asarray(p, dtype=np.int32)
    except Exception:
        return None


_PERM = _perm_np()


def _perm_idx():
    if _PERM is not None:
        return jnp.asarray(_PERM)
    return jax.random.permutation(jax.random.key(42), N).astype(jnp.int32)


def _sc_body(in_hbm, idx_hbm, out_hbm,
             row_v, idx_v0, idx_v1, out_v0, out_v1,
             is0, is1, os0, os1):
    wid = lax.axis_index("s") * NC + lax.axis_index("c")
    idx_bufs = (idx_v0, idx_v1)
    out_bufs = (out_v0, out_v1)
    idx_sems = (is0, is1)
    out_sems = (os0, os1)

    def row_body(k, carry):
        r = wid * ROWS_PER_W + k
        base = r * N
        pltpu.async_copy(idx_hbm.at[pl.ds(0, CHUNK)], idx_v0, is0)
        pltpu.sync_copy(in_hbm.at[pl.ds(base, N)], row_v)

        def outer(j2, carry):
            for b in (0, 1):
                j = 2 * j2 + b
                nb = 1 - b

                @pl.when(j + 1 < NCHUNK)
                def _():
                    pltpu.async_copy(
                        idx_hbm.at[pl.ds((j + 1) * CHUNK, CHUNK)],
                        idx_bufs[nb], idx_sems[nb])

                pltpu.make_async_copy(
                    idx_hbm.at[pl.ds(0, CHUNK)], idx_bufs[b],
                    idx_sems[b]).wait()

                @pl.when(j2 > 0)
                def _():
                    pltpu.make_async_copy(
                        out_bufs[b], out_hbm.at[pl.ds(0, CHUNK)],
                        out_sems[b]).wait()

                for v in range(STEPS):
                    sl = pl.ds(v * LANES, LANES)
                    out_bufs[b][sl] = plsc.load_gather(row_v, [idx_bufs[b][sl]])

                pltpu.async_copy(
                    out_bufs[b], out_hbm.at[pl.ds(base + j * CHUNK, CHUNK)],
                    out_sems[b])
            return carry

        lax.fori_loop(0, NCHUNK // 2, outer, 0)
        pltpu.make_async_copy(out_v0, out_hbm.at[pl.ds(0, CHUNK)], os0).wait()
        pltpu.make_async_copy(out_v1, out_hbm.at[pl.ds(0, CHUNK)], os1).wait()
        return carry

    lax.fori_loop(0, ROWS_PER_W, row_body, 0)


_shuffle_sc = pl.kernel(
    _sc_body,
    out_type=jax.ShapeDtypeStruct((R * N,), jnp.float32),
    mesh=plsc.VectorSubcoreMesh(
        core_axis_name="c", subcore_axis_name="s", num_cores=NC, num_subcores=NS
    ),
    scratch_types=[
        pltpu.VMEM((N,), jnp.float32),
        pltpu.VMEM((CHUNK,), jnp.int32),
        pltpu.VMEM((CHUNK,), jnp.int32),
        pltpu.VMEM((CHUNK,), jnp.float32),
        pltpu.VMEM((CHUNK,), jnp.float32),
        pltpu.SemaphoreType.DMA,
        pltpu.SemaphoreType.DMA,
        pltpu.SemaphoreType.DMA,
        pltpu.SemaphoreType.DMA,
    ],
    compiler_params=pltpu.CompilerParams(needs_layout_passes=False),
)


@jax.jit
def _run(inputs):
    idx = _perm_idx()
    flat = _shuffle_sc(inputs.reshape(R * N), idx)
    return flat.reshape(R, N)


def kernel(inputs):
    return _run(inputs)


# speedup vs baseline: 1.3078x; 1.0938x over previous
import jax
import jax.numpy as jnp
import numpy as np
from jax import lax
from jax.experimental import pallas as pl
from jax.experimental.pallas import tpu as pltpu
from jax.experimental.pallas import tpu_sc as plsc

R = 128
N = 100000
NC = 2
NS = 16
NW = NC * NS
ROWS_PER_W = R // NW
CHUNK = 2000
NCHUNK = N // CHUNK
LANES = 16
STEPS = CHUNK // LANES


def _perm_np():
    try:
        with jax.default_device(jax.local_devices(backend="cpu")[0]):
            p = jax.random.permutation(jax.random.key(42), N)
            return np.asarray(p, dtype=np.int32)
    except Exception:
        return None


_PERM = _perm_np()


def _perm_idx():
    if _PERM is not None:
        return jnp.asarray(_PERM)
    return jax.random.permutation(jax.random.key(42), N).astype(jnp.int32)


def _sc_body(in_hbm, idx_hbm, out_hbm,
             row_v, idx_v0, idx_v1, out_v0, out_v1, idx_sp,
             is0, is1, os0, os1):
    sid = lax.axis_index("s")
    wid = sid * NC + lax.axis_index("c")
    idx_bufs = (idx_v0, idx_v1)
    out_bufs = (out_v0, out_v1)
    idx_sems = (is0, is1)
    out_sems = (os0, os1)

    @pl.when(sid == 0)
    def _():
        pltpu.sync_copy(idx_hbm, idx_sp)

    plsc.subcore_barrier()

    def row_body(k, carry):
        r = wid * ROWS_PER_W + k
        base = r * N
        pltpu.async_copy(idx_sp.at[pl.ds(0, CHUNK)], idx_v0, is0)
        pltpu.sync_copy(in_hbm.at[pl.ds(base, N)], row_v)

        def outer(j2, carry):
            for b in (0, 1):
                j = 2 * j2 + b
                nb = 1 - b

                @pl.when(j + 1 < NCHUNK)
                def _():
                    pltpu.async_copy(
                        idx_sp.at[pl.ds((j + 1) * CHUNK, CHUNK)],
                        idx_bufs[nb], idx_sems[nb])

                pltpu.make_async_copy(
                    idx_sp.at[pl.ds(0, CHUNK)], idx_bufs[b],
                    idx_sems[b]).wait()

                @pl.when(j2 > 0)
                def _():
                    pltpu.make_async_copy(
                        out_bufs[b], out_hbm.at[pl.ds(0, CHUNK)],
                        out_sems[b]).wait()

                for v in range(STEPS):
                    sl = pl.ds(v * LANES, LANES)
                    out_bufs[b][sl] = plsc.load_gather(row_v, [idx_bufs[b][sl]])

                pltpu.async_copy(
                    out_bufs[b], out_hbm.at[pl.ds(base + j * CHUNK, CHUNK)],
                    out_sems[b])
            return carry

        lax.fori_loop(0, NCHUNK // 2, outer, 0)
        pltpu.make_async_copy(out_v0, out_hbm.at[pl.ds(0, CHUNK)], os0).wait()
        pltpu.make_async_copy(out_v1, out_hbm.at[pl.ds(0, CHUNK)], os1).wait()
        return carry

    lax.fori_loop(0, ROWS_PER_W, row_body, 0)


_shuffle_sc = pl.kernel(
    _sc_body,
    out_type=jax.ShapeDtypeStruct((R * N,), jnp.float32),
    mesh=plsc.VectorSubcoreMesh(
        core_axis_name="c", subcore_axis_name="s", num_cores=NC, num_subcores=NS
    ),
    scratch_types=[
        pltpu.VMEM((N,), jnp.float32),
        pltpu.VMEM((CHUNK,), jnp.int32),
        pltpu.VMEM((CHUNK,), jnp.int32),
        pltpu.VMEM((CHUNK,), jnp.float32),
        pltpu.VMEM((CHUNK,), jnp.float32),
        pltpu.VMEM_SHARED((N,), jnp.int32),
        pltpu.SemaphoreType.DMA,
        pltpu.SemaphoreType.DMA,
        pltpu.SemaphoreType.DMA,
        pltpu.SemaphoreType.DMA,
    ],
    compiler_params=pltpu.CompilerParams(needs_layout_passes=False),
)


@jax.jit
def _run(inputs):
    idx = _perm_idx()
    flat = _shuffle_sc(inputs.reshape(R * N), idx)
    return flat.reshape(R, N)


def kernel(inputs):
    return _run(inputs)
